# baseline (speedup 1.0000x reference)
import functools

import jax
import jax.numpy as jnp
from jax import lax
from jax.experimental import pallas as pl
from jax.experimental.pallas import tpu as pltpu
from jax.experimental.pallas import tpu_sc as plsc

FEAT = 128
BATCH = 16384
MOM = 0.3
HALF = BATCH // 2

_info = plsc.get_sparse_core_info()
_NC, _NS = _info.num_cores, _info.num_subcores
_NW = _NC * _NS
_B_PER_W = HALF // _NW
_CHUNK = 128
_PAIRS = _CHUNK // 2


def _pack_pairs(src, dst):
    def pair(p, carry):
        for g in range(FEAT // 16):
            a = src[2 * p, pl.ds(g * 16, 16)]
            b = src[2 * p + 1, pl.ds(g * 16, 16)]
            au = lax.bitcast_convert_type(a, jnp.uint32)
            bu = lax.bitcast_convert_type(b, jnp.uint32)
            w = ((au + 0x8000) >> 16) | ((bu + 0x8000) & jnp.uint32(0xFFFF0000))
            dst[p, pl.ds(g * 16, 16)] = lax.bitcast_convert_type(w, jnp.float32)
        return carry

    lax.fori_loop(0, _PAIRS, pair, 0)


def _sc_gather_body(half_off, idx_hbm, mu_hbm, tau_hbm, mu_out, tau_out,
                    idx0, idx1, mu_a, tau_a, mu_b, tau_b,
                    mu_w0, tau_w0, mu_w1, tau_w1,
                    sem_ga, sem_gb, sem_s):
    wid = lax.axis_index("s") * _NC + lax.axis_index("c")
    base = wid * _B_PER_W
    src = half_off + base
    wbase = pl.multiple_of(base // 2, _PAIRS)
    pltpu.sync_copy(idx_hbm.at[pl.ds(src, _CHUNK)], idx0)
    g0m = pltpu.async_copy(mu_hbm.at[idx0], mu_a, sem_ga)
    g0t = pltpu.async_copy(tau_hbm.at[idx0], tau_a, sem_ga)
    pltpu.sync_copy(idx_hbm.at[pl.ds(src + _CHUNK, _CHUNK)], idx1)
    g1m = pltpu.async_copy(mu_hbm.at[idx1], mu_b, sem_gb)
    g1t = pltpu.async_copy(tau_hbm.at[idx1], tau_b, sem_gb)
    g0m.wait(); g0t.wait()
    _pack_pairs(mu_a, mu_w0)
    _pack_pairs(tau_a, tau_w0)
    s0m = pltpu.async_copy(mu_w0, mu_out.at[pl.ds(wbase, _PAIRS)], sem_s)
    s0t = pltpu.async_copy(tau_w0, tau_out.at[pl.ds(wbase, _PAIRS)], sem_s)
    g1m.wait(); g1t.wait()
    _pack_pairs(mu_b, mu_w1)
    _pack_pairs(tau_b, tau_w1)
    s1m = pltpu.async_copy(mu_w1, mu_out.at[pl.ds(wbase + _PAIRS, _PAIRS)], sem_s)
    s1t = pltpu.async_copy(tau_w1, tau_out.at[pl.ds(wbase + _PAIRS, _PAIRS)], sem_s)
    s0m.wait(); s0t.wait(); s1m.wait(); s1t.wait()


def _make_sc_gather(half_off):
    return functools.partial(
        pl.kernel,
        mesh=plsc.VectorSubcoreMesh(core_axis_name="c", subcore_axis_name="s"),
        out_type=[
            jax.ShapeDtypeStruct((HALF // 2, FEAT), jnp.float32),
            jax.ShapeDtypeStruct((HALF // 2, FEAT), jnp.float32),
        ],
        scratch_types=[
            pltpu.VMEM((_CHUNK,), jnp.int32),
            pltpu.VMEM((_CHUNK,), jnp.int32),
            pltpu.VMEM((_CHUNK, FEAT), jnp.float32),
            pltpu.VMEM((_CHUNK, FEAT), jnp.float32),
            pltpu.VMEM((_CHUNK, FEAT), jnp.float32),
            pltpu.VMEM((_CHUNK, FEAT), jnp.float32),
            pltpu.VMEM((_PAIRS, FEAT), jnp.float32),
            pltpu.VMEM((_PAIRS, FEAT), jnp.float32),
            pltpu.VMEM((_PAIRS, FEAT), jnp.float32),
            pltpu.VMEM((_PAIRS, FEAT), jnp.float32),
            pltpu.SemaphoreType.DMA,
            pltpu.SemaphoreType.DMA,
            pltpu.SemaphoreType.DMA,
        ],
    )(functools.partial(_sc_gather_body, half_off))


_sc_gather_h0 = _make_sc_gather(0)
_sc_gather_h1 = _make_sc_gather(HALF)

_BLK = 4096
_HBLKS = HALF // _BLK


def _norm(z, mu_a, sigma):
    z_hat = (z - mu_a) / sigma
    return (1.0 - MOM) * z_hat + MOM * z


def _decode(w_ref):
    wu = lax.bitcast_convert_type(w_ref[...], jnp.uint32)
    even = lax.bitcast_convert_type(wu << 16, jnp.float32)
    odd = lax.bitcast_convert_type(wu & jnp.uint32(0xFFFF0000), jnp.float32)
    return even, odd


def _tc_norm0_body(z_ref, mu_ref, tau_ref, o_ref):
    z3 = z_ref[...]
    mu_e, mu_o = _decode(mu_ref)
    tau_e, tau_o = _decode(tau_ref)
    sig_e = jnp.log1p(jnp.exp(tau_e))
    sig_o = jnp.log1p(jnp.exp(tau_o))
    oe = _norm(z3[:, 0, :], mu_e, sig_e)
    oo = _norm(z3[:, 1, :], mu_o, sig_o)
    o_ref[...] = jnp.stack([oe, oo], axis=1)


def _tc_norm1_body(z_ref, mu_ref, tau_ref, prev_ref, o_ref):
    _tc_norm0_body(z_ref, mu_ref, tau_ref, o_ref)


def _tc_norm0(z3, mu_w, tau_w):
    z_spec = pl.BlockSpec((_BLK // 2, 2, FEAT), lambda i: (i, 0, 0))
    w_spec = pl.BlockSpec((_BLK // 2, FEAT), lambda i: (i, 0))
    return pl.pallas_call(
        _tc_norm0_body,
        grid=(_HBLKS,),
        in_specs=[z_spec, w_spec, w_spec],
        out_specs=z_spec,
        out_shape=jax.ShapeDtypeStruct((BATCH // 2, 2, FEAT), jnp.float32),
    )(z3, mu_w, tau_w)


def _tc_norm1(z3, mu_w, tau_w, prev):
    z_spec = pl.BlockSpec((_BLK // 2, 2, FEAT), lambda i: (i + _HBLKS, 0, 0))
    w_spec = pl.BlockSpec((_BLK // 2, FEAT), lambda i: (i, 0))
    any_spec = pl.BlockSpec(memory_space=pl.ANY)
    return pl.pallas_call(
        _tc_norm1_body,
        grid=(_HBLKS,),
        in_specs=[z_spec, w_spec, w_spec, any_spec],
        out_specs=z_spec,
        out_shape=jax.ShapeDtypeStruct((BATCH // 2, 2, FEAT), jnp.float32),
        input_output_aliases={3: 0},
    )(z3, mu_w, tau_w, prev)


def kernel(z, a, mu, tau):
    a32 = a.astype(jnp.int32)
    mu_w0, tau_w0 = _sc_gather_h0(a32, mu, tau)
    mu_w1, tau_w1 = _sc_gather_h1(a32, mu, tau)
    z3 = z.reshape(BATCH // 2, 2, FEAT)
    out = _tc_norm0(z3, mu_w0, tau_w0)
    out = _tc_norm1(z3, mu_w1, tau_w1, out)
    return out.reshape(BATCH, FEAT)


# speedup vs baseline: 1.5435x; 1.5435x over previous
import functools

import jax
import jax.numpy as jnp
from jax import lax
from jax.experimental import pallas as pl
from jax.experimental.pallas import tpu as pltpu
from jax.experimental.pallas import tpu_sc as plsc

FEAT = 128
BATCH = 16384
MOM = 0.3
SPLIT0 = 10240
SPLIT1 = BATCH - SPLIT0

_info = plsc.get_sparse_core_info()
_NC, _NS = _info.num_cores, _info.num_subcores
_NW = _NC * _NS
_CHUNK = 128


def _make_sc_gather(start, nrows):
    per_w = nrows // _NW
    chunks = []
    o = 0
    while o < per_w:
        c = min(_CHUNK, per_w - o)
        chunks.append((o, c))
        o += c
    n = len(chunks)

    def body(idx_hbm, mu_hbm, tau_hbm, mu_out, tau_out, *rest):
        idxs = rest[:n]
        bufs = rest[n:3 * n]
        gsems = rest[3 * n:4 * n]
        ssem = rest[-1]
        wid = lax.axis_index("s") * _NC + lax.axis_index("c")
        base = pl.multiple_of(wid * per_w, 8)
        gs = []
        for i, (o, c) in enumerate(chunks):
            pltpu.sync_copy(idx_hbm.at[pl.ds(start + base + o, c)], idxs[i])
            gm = pltpu.async_copy(mu_hbm.at[idxs[i]], bufs[2 * i], gsems[i])
            gt = pltpu.async_copy(tau_hbm.at[idxs[i]], bufs[2 * i + 1], gsems[i])
            gs.append((gm, gt))
        scatters = []
        for i, (o, c) in enumerate(chunks):
            gs[i][0].wait()
            gs[i][1].wait()
            scatters.append(pltpu.async_copy(
                bufs[2 * i], mu_out.at[pl.ds(base + o, c)], ssem))
            scatters.append(pltpu.async_copy(
                bufs[2 * i + 1], tau_out.at[pl.ds(base + o, c)], ssem))
        for s in scatters:
            s.wait()

    scratch = [pltpu.VMEM((c,), jnp.int32) for _, c in chunks]
    for _, c in chunks:
        scratch.append(pltpu.VMEM((c, FEAT), jnp.float32))
        scratch.append(pltpu.VMEM((c, FEAT), jnp.float32))
    scratch += [pltpu.SemaphoreType.DMA] * (n + 1)

    return functools.partial(
        pl.kernel,
        mesh=plsc.VectorSubcoreMesh(core_axis_name="c", subcore_axis_name="s"),
        out_type=[
            jax.ShapeDtypeStruct((nrows, FEAT), jnp.float32),
            jax.ShapeDtypeStruct((nrows, FEAT), jnp.float32),
        ],
        scratch_types=scratch,
    )(body)


_sc_gather_s0 = _make_sc_gather(0, SPLIT0)
_sc_gather_s1 = _make_sc_gather(SPLIT0, SPLIT1)

_BLK = 2048
_BLKS0 = SPLIT0 // _BLK
_BLKS1 = SPLIT1 // _BLK


def _tc_norm0_body(z_ref, mu_ref, tau_ref, o_ref):
    z = z_ref[...]
    sigma = jnp.log1p(jnp.exp(tau_ref[...]))
    z_hat = (z - mu_ref[...]) / sigma
    o_ref[...] = (1.0 - MOM) * z_hat + MOM * z


def _tc_norm1_body(z_ref, mu_ref, tau_ref, prev_ref, o_ref):
    _tc_norm0_body(z_ref, mu_ref, tau_ref, o_ref)


def _tc_norm0(z, mu_a, tau_a):
    spec = pl.BlockSpec((_BLK, FEAT), lambda i: (i, 0))
    return pl.pallas_call(
        _tc_norm0_body,
        grid=(_BLKS0,),
        in_specs=[spec, spec, spec],
        out_specs=spec,
        out_shape=jax.ShapeDtypeStruct((BATCH, FEAT), jnp.float32),
    )(z, mu_a, tau_a)


def _tc_norm1(z, mu_a, tau_a, prev):
    spec = pl.BlockSpec((_BLK, FEAT), lambda i: (i, 0))
    off_spec = pl.BlockSpec((_BLK, FEAT), lambda i: (i + _BLKS0, 0))
    any_spec = pl.BlockSpec(memory_space=pl.ANY)
    return pl.pallas_call(
        _tc_norm1_body,
        grid=(_BLKS1,),
        in_specs=[off_spec, spec, spec, any_spec],
        out_specs=off_spec,
        out_shape=jax.ShapeDtypeStruct((BATCH, FEAT), jnp.float32),
        input_output_aliases={3: 0},
    )(z, mu_a, tau_a, prev)


def kernel(z, a, mu, tau):
    a32 = a.astype(jnp.int32)
    mu_a0, tau_a0 = _sc_gather_s0(a32, mu, tau)
    mu_a1, tau_a1 = _sc_gather_s1(a32, mu, tau)
    out = _tc_norm0(z, mu_a0, tau_a0)
    out = _tc_norm1(z, mu_a1, tau_a1, out)
    return out


# speedup vs baseline: 1.5873x; 1.0284x over previous
import functools

import jax
import jax.numpy as jnp
from jax import lax
from jax.experimental import pallas as pl
from jax.experimental.pallas import tpu as pltpu
from jax.experimental.pallas import tpu_sc as plsc

FEAT = 128
BATCH = 16384
MOM = 0.3
HALF = BATCH // 2

_info = plsc.get_sparse_core_info()
_NC, _NS = _info.num_cores, _info.num_subcores
_NW = _NC * _NS
_B_PER_W = HALF // _NW
_CHUNK = 128


def _sc_gather_body(half_off, idx_hbm, mu_hbm, tau_hbm, mu_out, tau_out,
                    idx0, idx1, mu_a, tau_a, mu_b, tau_b,
                    sem_ga, sem_gb, sem_s):
    wid = lax.axis_index("s") * _NC + lax.axis_index("c")
    base = wid * _B_PER_W
    src = half_off + base
    pltpu.sync_copy(idx_hbm.at[pl.ds(src, _CHUNK)], idx0)
    g0m = pltpu.async_copy(mu_hbm.at[idx0], mu_a, sem_ga)
    g0t = pltpu.async_copy(tau_hbm.at[idx0], tau_a, sem_ga)
    pltpu.sync_copy(idx_hbm.at[pl.ds(src + _CHUNK, _CHUNK)], idx1)
    g1m = pltpu.async_copy(mu_hbm.at[idx1], mu_b, sem_gb)
    g1t = pltpu.async_copy(tau_hbm.at[idx1], tau_b, sem_gb)
    g0m.wait(); g0t.wait()
    s0m = pltpu.async_copy(mu_a, mu_out.at[pl.ds(base, _CHUNK)], sem_s)
    s0t = pltpu.async_copy(tau_a, tau_out.at[pl.ds(base, _CHUNK)], sem_s)
    g1m.wait(); g1t.wait()
    s1m = pltpu.async_copy(mu_b, mu_out.at[pl.ds(base + _CHUNK, _CHUNK)], sem_s)
    s1t = pltpu.async_copy(tau_b, tau_out.at[pl.ds(base + _CHUNK, _CHUNK)], sem_s)
    s0m.wait(); s0t.wait(); s1m.wait(); s1t.wait()


def _make_sc_gather(half_off):
    return functools.partial(
        pl.kernel,
        mesh=plsc.VectorSubcoreMesh(core_axis_name="c", subcore_axis_name="s"),
        out_type=[
            jax.ShapeDtypeStruct((HALF, FEAT), jnp.float32),
            jax.ShapeDtypeStruct((HALF, FEAT), jnp.float32),
        ],
        scratch_types=[
            pltpu.VMEM((_CHUNK,), jnp.int32),
            pltpu.VMEM((_CHUNK,), jnp.int32),
            pltpu.VMEM((_CHUNK, FEAT), jnp.float32),
            pltpu.VMEM((_CHUNK, FEAT), jnp.float32),
            pltpu.VMEM((_CHUNK, FEAT), jnp.float32),
            pltpu.VMEM((_CHUNK, FEAT), jnp.float32),
            pltpu.SemaphoreType.DMA,
            pltpu.SemaphoreType.DMA,
            pltpu.SemaphoreType.DMA,
        ],
    )(functools.partial(_sc_gather_body, half_off))


_sc_gather_h0 = _make_sc_gather(0)
_sc_gather_h1 = _make_sc_gather(HALF)

_BLK = 4096
_HBLKS = HALF // _BLK


def _tc_norm0_body(z_ref, mu_ref, tau_ref, o_ref):
    z = z_ref[...]
    sigma = jnp.log1p(jnp.exp(tau_ref[...]))
    z_hat = (z - mu_ref[...]) / sigma
    o_ref[...] = (1.0 - MOM) * z_hat + MOM * z


def _tc_norm1_body(z_ref, mu_ref, tau_ref, prev_ref, o_ref):
    _tc_norm0_body(z_ref, mu_ref, tau_ref, o_ref)


def _tc_norm0(z, mu_a, tau_a):
    half_spec = pl.BlockSpec((_BLK, FEAT), lambda i: (i, 0))
    return pl.pallas_call(
        _tc_norm0_body,
        grid=(_HBLKS,),
        in_specs=[half_spec, half_spec, half_spec],
        out_specs=half_spec,
        out_shape=jax.ShapeDtypeStruct((BATCH, FEAT), jnp.float32),
    )(z, mu_a, tau_a)


def _tc_norm1(z, mu_a, tau_a, prev):
    half_spec = pl.BlockSpec((_BLK, FEAT), lambda i: (i, 0))
    off_spec = pl.BlockSpec((_BLK, FEAT), lambda i: (i + _HBLKS, 0))
    any_spec = pl.BlockSpec(memory_space=pl.ANY)
    return pl.pallas_call(
        _tc_norm1_body,
        grid=(_HBLKS,),
        in_specs=[off_spec, half_spec, half_spec, any_spec],
        out_specs=off_spec,
        out_shape=jax.ShapeDtypeStruct((BATCH, FEAT), jnp.float32),
        input_output_aliases={3: 0},
    )(z, mu_a, tau_a, prev)


def kernel(z, a, mu, tau):
    a32 = a.astype(jnp.int32)
    mu_a0, tau_a0 = _sc_gather_h0(a32, mu, tau)
    mu_a1, tau_a1 = _sc_gather_h1(a32, mu, tau)
    out = _tc_norm0(z, mu_a0, tau_a0)
    out = _tc_norm1(z, mu_a1, tau_a1, out)
    return out


# speedup vs baseline: 1.5916x; 1.0027x over previous
import functools

import jax
import jax.numpy as jnp
from jax import lax
from jax.experimental import pallas as pl
from jax.experimental.pallas import tpu as pltpu
from jax.experimental.pallas import tpu_sc as plsc

FEAT = 128
BATCH = 16384
MOM = 0.3
HALF = BATCH // 2

_info = plsc.get_sparse_core_info()
_NC, _NS = _info.num_cores, _info.num_subcores
_NW = _NC * _NS
_B_PER_W = HALF // _NW
_CHUNK = 128


def _sc_gather_body(half_off, idx_hbm, mu_hbm, tau_hbm, mu_out, tau_out,
                    idx0, idx1, mu_a, tau_a, mu_b, tau_b,
                    sem_ga, sem_gb, sem_s):
    wid = lax.axis_index("s") * _NC + lax.axis_index("c")
    base = wid * _B_PER_W
    src = half_off + base
    i0 = pltpu.async_copy(idx_hbm.at[pl.ds(src, _CHUNK)], idx0, sem_s)
    i1 = pltpu.async_copy(idx_hbm.at[pl.ds(src + _CHUNK, _CHUNK)], idx1, sem_s)
    i0.wait()
    g0m = pltpu.async_copy(mu_hbm.at[idx0], mu_a, sem_ga)
    g0t = pltpu.async_copy(tau_hbm.at[idx0], tau_a, sem_ga)
    i1.wait()
    g1m = pltpu.async_copy(mu_hbm.at[idx1], mu_b, sem_gb)
    g1t = pltpu.async_copy(tau_hbm.at[idx1], tau_b, sem_gb)
    g0m.wait(); g0t.wait()
    s0m = pltpu.async_copy(mu_a, mu_out.at[pl.ds(base, _CHUNK)], sem_s)
    s0t = pltpu.async_copy(tau_a, tau_out.at[pl.ds(base, _CHUNK)], sem_s)
    g1m.wait(); g1t.wait()
    s1m = pltpu.async_copy(mu_b, mu_out.at[pl.ds(base + _CHUNK, _CHUNK)], sem_s)
    s1t = pltpu.async_copy(tau_b, tau_out.at[pl.ds(base + _CHUNK, _CHUNK)], sem_s)
    s0m.wait(); s0t.wait(); s1m.wait(); s1t.wait()


def _make_sc_gather(half_off):
    return functools.partial(
        pl.kernel,
        mesh=plsc.VectorSubcoreMesh(core_axis_name="c", subcore_axis_name="s"),
        out_type=[
            jax.ShapeDtypeStruct((HALF, FEAT), jnp.float32),
            jax.ShapeDtypeStruct((HALF, FEAT), jnp.float32),
        ],
        scratch_types=[
            pltpu.VMEM((_CHUNK,), jnp.int32),
            pltpu.VMEM((_CHUNK,), jnp.int32),
            pltpu.VMEM((_CHUNK, FEAT), jnp.float32),
            pltpu.VMEM((_CHUNK, FEAT), jnp.float32),
            pltpu.VMEM((_CHUNK, FEAT), jnp.float32),
            pltpu.VMEM((_CHUNK, FEAT), jnp.float32),
            pltpu.SemaphoreType.DMA,
            pltpu.SemaphoreType.DMA,
            pltpu.SemaphoreType.DMA,
        ],
    )(functools.partial(_sc_gather_body, half_off))


_sc_gather_h0 = _make_sc_gather(0)
_sc_gather_h1 = _make_sc_gather(HALF)

_BLK = 4096
_HBLKS = HALF // _BLK


def _tc_norm0_body(z_ref, mu_ref, tau_ref, o_ref):
    z = z_ref[...]
    sigma = jnp.log1p(jnp.exp(tau_ref[...]))
    z_hat = (z - mu_ref[...]) / sigma
    o_ref[...] = (1.0 - MOM) * z_hat + MOM * z


def _tc_norm1_body(z_ref, mu_ref, tau_ref, prev_ref, o_ref):
    _tc_norm0_body(z_ref, mu_ref, tau_ref, o_ref)


def _tc_norm0(z, mu_a, tau_a):
    half_spec = pl.BlockSpec((_BLK, FEAT), lambda i: (i, 0))
    return pl.pallas_call(
        _tc_norm0_body,
        grid=(_HBLKS,),
        in_specs=[half_spec, half_spec, half_spec],
        out_specs=half_spec,
        out_shape=jax.ShapeDtypeStruct((BATCH, FEAT), jnp.float32),
    )(z, mu_a, tau_a)


def _tc_norm1(z, mu_a, tau_a, prev):
    half_spec = pl.BlockSpec((_BLK, FEAT), lambda i: (i, 0))
    off_spec = pl.BlockSpec((_BLK, FEAT), lambda i: (i + _HBLKS, 0))
    any_spec = pl.BlockSpec(memory_space=pl.ANY)
    return pl.pallas_call(
        _tc_norm1_body,
        grid=(_HBLKS,),
        in_specs=[off_spec, half_spec, half_spec, any_spec],
        out_specs=off_spec,
        out_shape=jax.ShapeDtypeStruct((BATCH, FEAT), jnp.float32),
        input_output_aliases={3: 0},
    )(z, mu_a, tau_a, prev)


def kernel(z, a, mu, tau):
    a32 = a.astype(jnp.int32)
    mu_a0, tau_a0 = _sc_gather_h0(a32, mu, tau)
    mu_a1, tau_a1 = _sc_gather_h1(a32, mu, tau)
    out = _tc_norm0(z, mu_a0, tau_a0)
    out = _tc_norm1(z, mu_a1, tau_a1, out)
    return out
